# baseline (speedup 1.0000x reference)
import jax
import jax.numpy as jnp
from jax import lax
from jax.experimental import pallas as pl
from jax.experimental.pallas import tpu as pltpu
from jax.experimental.pallas import tpu_sc as plsc

B = 16384
D = 64
NC = 2
NS = 16
NW = NC * NS
BPW = B // NW
RND = 2
BPR = BPW // RND
OPW = BPW * D // 128
OPR = OPW // RND
CHUNK = 128
LANES = 16


def _gmf_body(user_ids, movie_ids, flat_u, flat_m, out,
              idx_u, idx_m, widx_u, widx_m, gat_u, gat_m, out_v, sem_u, sem_m):
    wid = lax.axis_index("s") * NC + lax.axis_index("c")
    base = wid * BPW

    pltpu.sync_copy(user_ids.at[pl.ds(base, BPW)], idx_u)
    pltpu.sync_copy(movie_ids.at[pl.ds(base, BPW)], idx_m)

    lane = lax.iota(jnp.int32, LANES)

    for r in range(RND):
        def expand(g, carry):
            vu = idx_u[pl.ds(r * BPR + g * LANES, LANES)]
            vm = idx_m[pl.ds(r * BPR + g * LANES, LANES)]
            for j in range(LANES):
                su = vu[j] * D
                sm = vm[j] * D
                for c in range(D // LANES):
                    sl = pl.ds((g * LANES + j) * D + c * LANES, LANES)
                    widx_u[sl] = su + c * LANES + lane
                    widx_m[sl] = sm + c * LANES + lane
            return carry

        lax.fori_loop(0, BPR // LANES, expand, 0)

        def fire(k, carry):
            pltpu.async_copy(flat_u.at[widx_u.at[pl.ds(k * CHUNK, CHUNK)]],
                             gat_u.at[k], sem_u)
            pltpu.async_copy(flat_m.at[widx_m.at[pl.ds(k * CHUNK, CHUNK)]],
                             gat_m.at[k], sem_m)
            return carry

        lax.fori_loop(0, OPR, fire, 0)

        pltpu.make_async_copy(out.at[pl.ds(0, OPR)], gat_u, sem_u).wait()
        pltpu.make_async_copy(out.at[pl.ds(0, OPR)], gat_m, sem_m).wait()

        def mul_row(k, carry):
            orow = r * OPR + k
            for c in range(128 // LANES):
                sl = pl.ds(c * LANES, LANES)
                out_v[orow, sl] = gat_u[k, sl] * gat_m[k, sl]
            return carry

        lax.fori_loop(0, OPR, mul_row, 0)

    pltpu.sync_copy(out_v, out.at[pl.ds(wid * OPW, OPW)])


def kernel(user_ids, movie_ids, user_table, item_table):
    flat_u = user_table.reshape(-1)
    flat_m = item_table.reshape(-1)
    mesh = plsc.VectorSubcoreMesh(core_axis_name="c", subcore_axis_name="s")
    run = pl.kernel(
        _gmf_body,
        mesh=mesh,
        compiler_params=pltpu.CompilerParams(use_tc_tiling_on_sc=True),
        out_type=jax.ShapeDtypeStruct((B * D // 128, 128), jnp.float32),
        scratch_types=[
            pltpu.VMEM((BPW,), jnp.int32),
            pltpu.VMEM((BPW,), jnp.int32),
            pltpu.VMEM((BPR * D,), jnp.int32),
            pltpu.VMEM((BPR * D,), jnp.int32),
            pltpu.VMEM((OPR, 128), jnp.float32),
            pltpu.VMEM((OPR, 128), jnp.float32),
            pltpu.VMEM((OPW, 128), jnp.float32),
            pltpu.SemaphoreType.DMA,
            pltpu.SemaphoreType.DMA,
        ],
    )
    flat = run(user_ids.astype(jnp.int32), movie_ids.astype(jnp.int32),
               flat_u, flat_m)
    return flat.reshape(B, D)


# speedup vs baseline: 1.7043x; 1.7043x over previous
import jax
import jax.numpy as jnp
from jax import lax
from jax.experimental import pallas as pl
from jax.experimental.pallas import tpu as pltpu
from jax.experimental.pallas import tpu_sc as plsc

B = 16384
D = 64
NC = 2
NS = 16
NW = NC * NS
BPW = B // NW
OPW = BPW * D // 128
LANES = 16
NSEM = 4


def _gmf_body(user_ids, movie_ids, user_table, item_table, out,
              idx_uv, idx_mv, rows_u, rows_m, sem_u, sem_m):
    wid = lax.axis_index("s") * NC + lax.axis_index("c")
    base = wid * BPW

    pltpu.sync_copy(user_ids.at[pl.ds(base, BPW)], idx_uv)
    pltpu.sync_copy(movie_ids.at[pl.ds(base, BPW)], idx_mv)

    def fire(ch, carry):
        vu = idx_uv[pl.ds(ch * LANES, LANES)]
        vm = idx_mv[pl.ds(ch * LANES, LANES)]
        for j in range(LANES):
            q = ch * (LANES // 2) + j // 2
            h = (j % 2) * D
            pltpu.async_copy(user_table.at[vu[j]], rows_u.at[q, pl.ds(h, D)],
                             sem_u.at[j // 4])
            pltpu.async_copy(item_table.at[vm[j]], rows_m.at[q, pl.ds(h, D)],
                             sem_m.at[j // 4])
        return carry

    lax.fori_loop(0, BPW // LANES, fire, 0)

    def drain(r, carry):
        for s in range(NSEM):
            pltpu.make_async_copy(user_table.at[0], rows_u.at[0, pl.ds(0, D)],
                                  sem_u.at[s]).wait()
            pltpu.make_async_copy(item_table.at[0], rows_m.at[0, pl.ds(0, D)],
                                  sem_m.at[s]).wait()
        return carry

    lax.fori_loop(0, BPW // NSEM, drain, 0)

    def mul_row(i, carry):
        for c in range(128 // LANES):
            sl = pl.ds(c * LANES, LANES)
            rows_u[i, sl] = rows_u[i, sl] * rows_m[i, sl]
        return carry

    lax.fori_loop(0, OPW, mul_row, 0)

    pltpu.sync_copy(rows_u, out.at[pl.ds(wid * OPW, OPW)])


def kernel(user_ids, movie_ids, user_table, item_table):
    mesh = plsc.VectorSubcoreMesh(core_axis_name="c", subcore_axis_name="s")
    run = pl.kernel(
        _gmf_body,
        mesh=mesh,
        compiler_params=pltpu.CompilerParams(use_tc_tiling_on_sc=True),
        out_type=jax.ShapeDtypeStruct((B * D // 128, 128), jnp.float32),
        scratch_types=[
            pltpu.VMEM((BPW,), jnp.int32),
            pltpu.VMEM((BPW,), jnp.int32),
            pltpu.VMEM((OPW, 128), jnp.float32),
            pltpu.VMEM((OPW, 128), jnp.float32),
            pltpu.SemaphoreType.DMA((NSEM,)),
            pltpu.SemaphoreType.DMA((NSEM,)),
        ],
    )
    flat = run(user_ids.astype(jnp.int32), movie_ids.astype(jnp.int32),
               user_table, item_table)
    return flat.reshape(B, D)
